# baseline (speedup 1.0000x reference)
import functools

import jax
import jax.numpy as jnp
from jax import lax
from jax.experimental import pallas as pl
from jax.experimental.pallas import tpu as pltpu
from jax.experimental.pallas import tpu_sc as plsc

N, D, E = 2048, 768, 64
TB = 128
BLK = 128
NC, NS = 2, 16
NW = NC * NS
CH = N // NW


def _router_body(x_ref, wr_ref, br_ref, pos_ref, st_ref, en_ref):
    nb = N // TB
    logits = jnp.dot(x_ref[...], wr_ref[...],
                     preferred_element_type=jnp.float32) + br_ref[...]
    m = jnp.max(logits, axis=1, keepdims=True)
    iota_e = lax.broadcasted_iota(jnp.int32, (N, E), 1)
    sel = jnp.min(jnp.where(logits >= m, iota_e, E), axis=1, keepdims=True)
    onehot = (iota_e == sel).astype(jnp.float32)

    rB = lax.broadcasted_iota(jnp.int32, (nb, N), 0)
    cB = lax.broadcasted_iota(jnp.int32, (nb, N), 1)
    bmat = (cB // TB == rB).astype(jnp.bfloat16)
    bc = jnp.dot(bmat, onehot.astype(jnp.bfloat16),
                 preferred_element_type=jnp.float32)
    rT = lax.broadcasted_iota(jnp.int32, (nb, nb), 0)
    cT = lax.broadcasted_iota(jnp.int32, (nb, nb), 1)
    base = jnp.dot((cT < rT).astype(jnp.float32), bc,
                   preferred_element_type=jnp.float32)
    counts = jnp.sum(bc, axis=0, keepdims=True)
    r64 = lax.broadcasted_iota(jnp.int32, (E, E), 0)
    c64 = lax.broadcasted_iota(jnp.int32, (E, E), 1)
    excl = jnp.dot(counts, (r64 < c64).astype(jnp.float32),
                   preferred_element_type=jnp.float32)
    st_ref[...] = excl.astype(jnp.int32)
    en_ref[...] = (excl + counts).astype(jnp.int32)

    r128 = lax.broadcasted_iota(jnp.int32, (TB, TB), 0)
    c128 = lax.broadcasted_iota(jnp.int32, (TB, TB), 1)
    tril = (r128 >= c128).astype(jnp.bfloat16)
    for t in range(nb):
        oh_t = onehot[t * TB:(t + 1) * TB, :]
        cum_t = jnp.dot(tril, oh_t.astype(jnp.bfloat16),
                        preferred_element_type=jnp.float32) + base[t:t + 1, :]
        rk_off = jnp.sum(oh_t * (cum_t + excl), axis=1, keepdims=True)
        pos_ref[t * TB:(t + 1) * TB, :] = (rk_off - 1.0).astype(jnp.int32)


_router = pl.pallas_call(
    _router_body,
    out_shape=(jax.ShapeDtypeStruct((N, 1), jnp.int32),
               jax.ShapeDtypeStruct((1, E), jnp.int32),
               jax.ShapeDtypeStruct((1, E), jnp.int32)),
)


EPB = 8


def _moe_body(starts_ref, ends_ref, x_ref, w_ref, b_ref, out_ref):
    g = pl.program_id(0)

    for j in range(EPB):
        e = g * EPB + j
        start = starts_ref[e]
        end = ends_ref[e]
        w = w_ref[j]
        b = b_ref[j]

        def chunk(c, _, start=start, end=end, w=w, b=b):
            rb = c * BLK
            xa = x_ref[pl.ds(rb, BLK), :]
            res = jnp.dot(xa, w, preferred_element_type=jnp.float32) + b
            rows = rb + lax.broadcasted_iota(jnp.int32, (BLK, 1), 0)
            masked = jnp.where((rows >= start) & (rows < end), res, 0.0)

            @pl.when(start <= rb)
            def _():
                out_ref[pl.ds(rb, BLK), :] = masked

            @pl.when(start > rb)
            def _():
                out_ref[pl.ds(rb, BLK), :] = out_ref[pl.ds(rb, BLK), :] + masked

            return 0

        lax.fori_loop(start // BLK, (end - 1) // BLK + 1, chunk, 0)


_moe = pl.pallas_call(
    _moe_body,
    grid_spec=pltpu.PrefetchScalarGridSpec(
        num_scalar_prefetch=2,
        grid=(E // EPB,),
        in_specs=[
            pl.BlockSpec((N, D), lambda g, s, t: (0, 0)),
            pl.BlockSpec((EPB, D, D), lambda g, s, t: (g, 0, 0)),
            pl.BlockSpec((EPB, 1, D), lambda g, s, t: (g, 0, 0)),
        ],
        out_specs=pl.BlockSpec((N, D), lambda g, s, t: (0, 0)),
    ),
    out_shape=jax.ShapeDtypeStruct((N, D), jnp.float32),
)


@functools.cache
def _make_permute(mode):
    @functools.partial(
        pl.kernel,
        mesh=plsc.VectorSubcoreMesh(core_axis_name="c", subcore_axis_name="s"),
        out_type=jax.ShapeDtypeStruct((N, D), jnp.float32),
        scratch_types=[
            pltpu.VMEM((CH,), jnp.int32),
            pltpu.VMEM((CH, D), jnp.float32),
            pltpu.SemaphoreType.DMA,
            pltpu.SemaphoreType.DMA,
        ],
    )
    def k(src, idx, out, idx_v, rows_v, sem, sem2):
        wid = lax.axis_index("s") * NC + lax.axis_index("c")
        base = wid * CH
        if mode == "scatter":
            c1 = pltpu.async_copy(idx.at[pl.ds(base, CH)], idx_v, sem)
            c2 = pltpu.async_copy(src.at[pl.ds(base, CH)], rows_v, sem2)
            c1.wait()
            c2.wait()
            pltpu.async_copy(rows_v, out.at[idx_v], sem).wait()
        else:
            pltpu.sync_copy(idx.at[pl.ds(base, CH)], idx_v)
            pltpu.async_copy(src.at[idx_v], rows_v, sem).wait()
            pltpu.sync_copy(rows_v, out.at[pl.ds(base, CH)])

    return k


def kernel(x, Wr, br, We, be):
    pos2d, st, en = _router(x, Wr, br.reshape(1, E))
    pos = pos2d.reshape(N)
    x_sorted = _make_permute("scatter")(x, pos)
    out_sorted = _moe(st.reshape(E), en.reshape(E), x_sorted, We,
                      be.reshape(E, 1, D))
    return _make_permute("gather")(out_sorted, pos)


# speedup vs baseline: 1.0179x; 1.0179x over previous
import functools

import jax
import jax.numpy as jnp
from jax import lax
from jax.experimental import pallas as pl
from jax.experimental.pallas import tpu as pltpu
from jax.experimental.pallas import tpu_sc as plsc

N, D, E = 2048, 768, 64
TB = 128
BLK = 128
NC, NS = 2, 16
NW = NC * NS
CH = N // NW


def _router_body(x_ref, wr_ref, br_ref, pos_ref, st_ref, en_ref):
    nb = N // TB
    logits = jnp.dot(x_ref[...], wr_ref[...],
                     preferred_element_type=jnp.float32) + br_ref[...]
    m = jnp.max(logits, axis=1, keepdims=True)
    iota_e = lax.broadcasted_iota(jnp.int32, (N, E), 1)
    sel = jnp.min(jnp.where(logits >= m, iota_e, E), axis=1, keepdims=True)
    onehot = (iota_e == sel).astype(jnp.float32)

    rB = lax.broadcasted_iota(jnp.int32, (nb, N), 0)
    cB = lax.broadcasted_iota(jnp.int32, (nb, N), 1)
    bmat = (cB // TB == rB).astype(jnp.bfloat16)
    bc = jnp.dot(bmat, onehot.astype(jnp.bfloat16),
                 preferred_element_type=jnp.float32)
    rT = lax.broadcasted_iota(jnp.int32, (nb, nb), 0)
    cT = lax.broadcasted_iota(jnp.int32, (nb, nb), 1)
    base = jnp.dot((cT < rT).astype(jnp.float32), bc,
                   preferred_element_type=jnp.float32)
    counts = jnp.sum(bc, axis=0, keepdims=True)
    r64 = lax.broadcasted_iota(jnp.int32, (E, E), 0)
    c64 = lax.broadcasted_iota(jnp.int32, (E, E), 1)
    excl = jnp.dot(counts, (r64 < c64).astype(jnp.float32),
                   preferred_element_type=jnp.float32)
    st_ref[...] = excl.astype(jnp.int32)
    en_ref[...] = (excl + counts).astype(jnp.int32)

    r128 = lax.broadcasted_iota(jnp.int32, (TB, TB), 0)
    c128 = lax.broadcasted_iota(jnp.int32, (TB, TB), 1)
    tril = (r128 >= c128).astype(jnp.bfloat16)
    for t in range(nb):
        oh_t = onehot[t * TB:(t + 1) * TB, :]
        cum_t = jnp.dot(tril, oh_t.astype(jnp.bfloat16),
                        preferred_element_type=jnp.float32) + base[t:t + 1, :]
        rk_off = jnp.sum(oh_t * (cum_t + excl), axis=1, keepdims=True)
        pos_ref[t * TB:(t + 1) * TB, :] = (rk_off - 1.0).astype(jnp.int32)


_router = pl.pallas_call(
    _router_body,
    out_shape=(jax.ShapeDtypeStruct((N, 1), jnp.int32),
               jax.ShapeDtypeStruct((1, E), jnp.int32),
               jax.ShapeDtypeStruct((1, E), jnp.int32)),
)


EPB = 4


def _moe_body(starts_ref, ends_ref, x_ref, w_ref, b_ref, out_ref):
    g = pl.program_id(0)

    for j in range(EPB):
        e = g * EPB + j
        start = starts_ref[e]
        end = ends_ref[e]
        w = w_ref[j]
        b = b_ref[j]

        def chunk(c, _, start=start, end=end, w=w, b=b):
            rb = c * BLK
            xa = x_ref[pl.ds(rb, BLK), :]
            res = jnp.dot(xa, w, preferred_element_type=jnp.float32) + b
            rows = rb + lax.broadcasted_iota(jnp.int32, (BLK, 1), 0)
            masked = jnp.where((rows >= start) & (rows < end), res, 0.0)

            @pl.when(start <= rb)
            def _():
                out_ref[pl.ds(rb, BLK), :] = masked

            @pl.when(start > rb)
            def _():
                out_ref[pl.ds(rb, BLK), :] = out_ref[pl.ds(rb, BLK), :] + masked

            return 0

        lax.fori_loop(start // BLK, (end - 1) // BLK + 1, chunk, 0)


_moe = pl.pallas_call(
    _moe_body,
    grid_spec=pltpu.PrefetchScalarGridSpec(
        num_scalar_prefetch=2,
        grid=(E // EPB,),
        in_specs=[
            pl.BlockSpec((N, D), lambda g, s, t: (0, 0)),
            pl.BlockSpec((EPB, D, D), lambda g, s, t: (g, 0, 0)),
            pl.BlockSpec((EPB, 1, D), lambda g, s, t: (g, 0, 0)),
        ],
        out_specs=pl.BlockSpec((N, D), lambda g, s, t: (0, 0)),
    ),
    out_shape=jax.ShapeDtypeStruct((N, D), jnp.float32),
)


@functools.cache
def _make_permute(mode):
    @functools.partial(
        pl.kernel,
        mesh=plsc.VectorSubcoreMesh(core_axis_name="c", subcore_axis_name="s"),
        out_type=jax.ShapeDtypeStruct((N, D), jnp.float32),
        scratch_types=[
            pltpu.VMEM((CH // 2,), jnp.int32),
            pltpu.VMEM((CH // 2,), jnp.int32),
            pltpu.VMEM((CH // 2, D), jnp.float32),
            pltpu.VMEM((CH // 2, D), jnp.float32),
            pltpu.SemaphoreType.DMA,
            pltpu.SemaphoreType.DMA,
            pltpu.SemaphoreType.DMA,
            pltpu.SemaphoreType.DMA,
        ],
    )
    def k(src, idx, out, i0, i1, r0, r1, s0, s1, s2, s3):
        wid = lax.axis_index("s") * NC + lax.axis_index("c")
        base = wid * CH
        H = CH // 2
        ci0 = pltpu.async_copy(idx.at[pl.ds(base, H)], i0, s0)
        ci1 = pltpu.async_copy(idx.at[pl.ds(base + H, H)], i1, s1)
        if mode == "scatter":
            cr0 = pltpu.async_copy(src.at[pl.ds(base, H)], r0, s2)
            cr1 = pltpu.async_copy(src.at[pl.ds(base + H, H)], r1, s3)
            ci0.wait()
            cr0.wait()
            w0 = pltpu.async_copy(r0, out.at[i0], s0)
            ci1.wait()
            cr1.wait()
            w1 = pltpu.async_copy(r1, out.at[i1], s1)
            w0.wait()
            w1.wait()
        else:
            ci0.wait()
            g0 = pltpu.async_copy(src.at[i0], r0, s2)
            ci1.wait()
            g1 = pltpu.async_copy(src.at[i1], r1, s3)
            g0.wait()
            w0 = pltpu.async_copy(r0, out.at[pl.ds(base, H)], s0)
            g1.wait()
            w1 = pltpu.async_copy(r1, out.at[pl.ds(base + H, H)], s1)
            w0.wait()
            w1.wait()

    return k


def kernel(x, Wr, br, We, be):
    pos2d, st, en = _router(x, Wr, br.reshape(1, E))
    pos = pos2d.reshape(N)
    x_sorted = _make_permute("scatter")(x, pos)
    out_sorted = _moe(st.reshape(E), en.reshape(E), x_sorted, We,
                      be.reshape(E, 1, D))
    return _make_permute("gather")(out_sorted, pos)
